# baseline (speedup 1.0000x reference)
import jax
import jax.numpy as jnp
from jax.experimental import pallas as pl
from jax.experimental.pallas import tpu as pltpu


def _elu(x):
    return jnp.where(x > 0, x, jnp.exp(jnp.minimum(x, 0.0)) - 1.0)


def _pre_kernel(f_ref, wds_ref, bds_ref, w0_ref, r0_ref, b0_ref,
                xw0_ref, xr0_ref):
    x = jnp.dot(f_ref[...].astype(jnp.bfloat16),
                wds_ref[...].astype(jnp.bfloat16),
                preferred_element_type=jnp.float32)
    x = _elu(x + bds_ref[...])
    xb = x.astype(jnp.bfloat16)
    w0 = w0_ref[...]
    for t in range(w0.shape[0]):
        xw0_ref[t] = jnp.dot(xb, w0[t].astype(jnp.bfloat16),
                             preferred_element_type=jnp.float32
                             ).astype(jnp.bfloat16)
    xr0_ref[...] = jnp.dot(xb, r0_ref[...].astype(jnp.bfloat16),
                           preferred_element_type=jnp.float32) + b0_ref[...]


def _gc_kernel(a0_ref, a1_ref, xw_ref, xr_ref, out_ref):
    acc = xr_ref[...]
    acc = acc + jnp.dot(a0_ref[0].astype(jnp.bfloat16), xw_ref[0],
                        preferred_element_type=jnp.float32)
    acc = acc + jnp.dot(a1_ref[0].astype(jnp.bfloat16), xw_ref[1],
                        preferred_element_type=jnp.float32)
    out_ref[...] = acc


def _gc2_kernel(a0_ref, a1_ref, h_ref, g_ref, be_ref, w1_ref, r1_ref, b1_ref,
                out_ref, xw_scr, hr_scr):
    i = pl.program_id(0)
    br = out_ref.shape[0]

    @pl.when(i == 0)
    def _mid():
        h = h_ref[...]
        mu = jnp.mean(h, axis=0, keepdims=True)
        var = jnp.mean((h - mu) ** 2, axis=0, keepdims=True)
        hn = (h - mu) * jax.lax.rsqrt(var + 1e-5) * g_ref[...] + be_ref[...]
        h1b = _elu(hn).astype(jnp.bfloat16)
        for t in range(w1_ref.shape[0]):
            xw_scr[t] = jnp.dot(h1b, w1_ref[t].astype(jnp.bfloat16),
                                preferred_element_type=jnp.float32
                                ).astype(jnp.bfloat16)
        hr_scr[...] = jnp.dot(h1b, r1_ref[...].astype(jnp.bfloat16),
                              preferred_element_type=jnp.float32) + b1_ref[...]

    acc = hr_scr[pl.ds(i * br, br), :]
    acc = acc + jnp.dot(a0_ref[0].astype(jnp.bfloat16), xw_scr[0],
                        preferred_element_type=jnp.float32)
    acc = acc + jnp.dot(a1_ref[0].astype(jnp.bfloat16), xw_scr[1],
                        preferred_element_type=jnp.float32)
    out_ref[...] = acc


def _gc_layer(adjacency, xw, xr, n, t_count, f_out, br):
    ni = n // br
    return pl.pallas_call(
        _gc_kernel,
        grid=(ni,),
        in_specs=[
            pl.BlockSpec((1, br, n), lambda i: (0, i, 0)),
            pl.BlockSpec((1, br, n), lambda i: (1, i, 0)),
            pl.BlockSpec((t_count, n, f_out), lambda i: (0, 0, 0)),
            pl.BlockSpec((br, f_out), lambda i: (i, 0)),
        ],
        out_specs=pl.BlockSpec((br, f_out), lambda i: (i, 0)),
        out_shape=jax.ShapeDtypeStruct((n, f_out), jnp.float32),
        compiler_params=pltpu.CompilerParams(
            dimension_semantics=("parallel",),
            vmem_limit_bytes=120 * 1024 * 1024,
        ),
    )(adjacency, adjacency, xw, xr)


def kernel(features, adjacency_matrix, W_ds, b_ds, W0, b0, R0,
           gamma1, beta1, W1, b1, R1):
    n, f_in = features.shape
    t_count = adjacency_matrix.shape[0]
    f_ds = W_ds.shape[1]
    f1 = W0.shape[2]
    f2 = W1.shape[2]

    br = 200 if n % 200 == 0 else n
    brp = 2000 if n % 2000 == 0 else n

    bds2 = b_ds.reshape(1, f_ds)
    b02 = b0.reshape(1, f1)
    b12 = b1.reshape(1, f2)
    g2 = gamma1.reshape(1, f1)
    be2 = beta1.reshape(1, f1)

    ni = n // brp
    xw0, xr0 = pl.pallas_call(
        _pre_kernel,
        grid=(ni,),
        in_specs=[
            pl.BlockSpec((brp, f_in), lambda i: (i, 0)),
            pl.BlockSpec((f_in, f_ds), lambda i: (0, 0)),
            pl.BlockSpec((1, f_ds), lambda i: (0, 0)),
            pl.BlockSpec((t_count, f_ds, f1), lambda i: (0, 0, 0)),
            pl.BlockSpec((f_ds, f1), lambda i: (0, 0)),
            pl.BlockSpec((1, f1), lambda i: (0, 0)),
        ],
        out_specs=[
            pl.BlockSpec((t_count, brp, f1), lambda i: (0, i, 0)),
            pl.BlockSpec((brp, f1), lambda i: (i, 0)),
        ],
        out_shape=[
            jax.ShapeDtypeStruct((t_count, n, f1), jnp.bfloat16),
            jax.ShapeDtypeStruct((n, f1), jnp.float32),
        ],
    )(features, W_ds, bds2, W0, R0, b02)

    h1raw = _gc_layer(adjacency_matrix, xw0, xr0, n, t_count, f1, br)

    return pl.pallas_call(
        _gc2_kernel,
        grid=(n // br,),
        in_specs=[
            pl.BlockSpec((1, br, n), lambda i: (0, i, 0)),
            pl.BlockSpec((1, br, n), lambda i: (1, i, 0)),
            pl.BlockSpec((n, f1), lambda i: (0, 0)),
            pl.BlockSpec((1, f1), lambda i: (0, 0)),
            pl.BlockSpec((1, f1), lambda i: (0, 0)),
            pl.BlockSpec((t_count, f1, f2), lambda i: (0, 0, 0)),
            pl.BlockSpec((f1, f2), lambda i: (0, 0)),
            pl.BlockSpec((1, f2), lambda i: (0, 0)),
        ],
        out_specs=pl.BlockSpec((br, f2), lambda i: (i, 0)),
        out_shape=jax.ShapeDtypeStruct((n, f2), jnp.float32),
        scratch_shapes=[
            pltpu.VMEM((t_count, n, f2), jnp.bfloat16),
            pltpu.VMEM((n, f2), jnp.float32),
        ],
        compiler_params=pltpu.CompilerParams(
            dimension_semantics=("arbitrary",),
            vmem_limit_bytes=120 * 1024 * 1024,
        ),
    )(adjacency_matrix, adjacency_matrix, h1raw, g2, be2, W1, R1, b12)


# speedup vs baseline: 1.0163x; 1.0163x over previous
import jax
import jax.numpy as jnp
from jax.experimental import pallas as pl
from jax.experimental.pallas import tpu as pltpu


def _elu(x):
    return jnp.where(x > 0, x, jnp.exp(jnp.minimum(x, 0.0)) - 1.0)


def _gc1_kernel(a0_ref, a1_ref, f_ref, wds_ref, bds_ref, w0_ref, r0_ref,
                b0_ref, out_ref, xw_scr, xr_scr):
    i = pl.program_id(0)
    br = out_ref.shape[0]

    @pl.when(i == 0)
    def _pre():
        n = f_ref.shape[0]
        step = 2000 if n % 2000 == 0 else n
        wds = wds_ref[...].astype(jnp.bfloat16)
        r0 = r0_ref[...].astype(jnp.bfloat16)
        for j in range(n // step):
            rows = pl.ds(j * step, step)
            x = jnp.dot(f_ref[rows, :].astype(jnp.bfloat16), wds,
                        preferred_element_type=jnp.float32)
            xb = _elu(x + bds_ref[...]).astype(jnp.bfloat16)
            for t in range(w0_ref.shape[0]):
                xw_scr[t, rows, :] = jnp.dot(
                    xb, w0_ref[t].astype(jnp.bfloat16),
                    preferred_element_type=jnp.float32).astype(jnp.bfloat16)
            xr_scr[rows, :] = jnp.dot(
                xb, r0, preferred_element_type=jnp.float32) + b0_ref[...]

    acc = xr_scr[pl.ds(i * br, br), :]
    acc = acc + jnp.dot(a0_ref[0].astype(jnp.bfloat16), xw_scr[0],
                        preferred_element_type=jnp.float32)
    acc = acc + jnp.dot(a1_ref[0].astype(jnp.bfloat16), xw_scr[1],
                        preferred_element_type=jnp.float32)
    out_ref[...] = acc


def _gc2_kernel(a0_ref, a1_ref, h_ref, g_ref, be_ref, w1_ref, r1_ref, b1_ref,
                out_ref, xw_scr, hr_scr):
    i = pl.program_id(0)
    br = out_ref.shape[0]

    @pl.when(i == 0)
    def _mid():
        h = h_ref[...]
        mu = jnp.mean(h, axis=0, keepdims=True)
        var = jnp.mean((h - mu) ** 2, axis=0, keepdims=True)
        hn = (h - mu) * jax.lax.rsqrt(var + 1e-5) * g_ref[...] + be_ref[...]
        h1b = _elu(hn).astype(jnp.bfloat16)
        for t in range(w1_ref.shape[0]):
            xw_scr[t] = jnp.dot(h1b, w1_ref[t].astype(jnp.bfloat16),
                                preferred_element_type=jnp.float32
                                ).astype(jnp.bfloat16)
        hr_scr[...] = jnp.dot(h1b, r1_ref[...].astype(jnp.bfloat16),
                              preferred_element_type=jnp.float32) + b1_ref[...]

    acc = hr_scr[pl.ds(i * br, br), :]
    acc = acc + jnp.dot(a0_ref[0].astype(jnp.bfloat16), xw_scr[0],
                        preferred_element_type=jnp.float32)
    acc = acc + jnp.dot(a1_ref[0].astype(jnp.bfloat16), xw_scr[1],
                        preferred_element_type=jnp.float32)
    out_ref[...] = acc


def kernel(features, adjacency_matrix, W_ds, b_ds, W0, b0, R0,
           gamma1, beta1, W1, b1, R1):
    n, f_in = features.shape
    t_count = adjacency_matrix.shape[0]
    f_ds = W_ds.shape[1]
    f1 = W0.shape[2]
    f2 = W1.shape[2]

    br = 200 if n % 200 == 0 else n
    ni = n // br

    bds2 = b_ds.reshape(1, f_ds)
    b02 = b0.reshape(1, f1)
    b12 = b1.reshape(1, f2)
    g2 = gamma1.reshape(1, f1)
    be2 = beta1.reshape(1, f1)

    h1raw = pl.pallas_call(
        _gc1_kernel,
        grid=(ni,),
        in_specs=[
            pl.BlockSpec((1, br, n), lambda i: (0, i, 0)),
            pl.BlockSpec((1, br, n), lambda i: (1, i, 0)),
            pl.BlockSpec((n, f_in), lambda i: (0, 0)),
            pl.BlockSpec((f_in, f_ds), lambda i: (0, 0)),
            pl.BlockSpec((1, f_ds), lambda i: (0, 0)),
            pl.BlockSpec((t_count, f_ds, f1), lambda i: (0, 0, 0)),
            pl.BlockSpec((f_ds, f1), lambda i: (0, 0)),
            pl.BlockSpec((1, f1), lambda i: (0, 0)),
        ],
        out_specs=pl.BlockSpec((br, f1), lambda i: (i, 0)),
        out_shape=jax.ShapeDtypeStruct((n, f1), jnp.float32),
        scratch_shapes=[
            pltpu.VMEM((t_count, n, f1), jnp.bfloat16),
            pltpu.VMEM((n, f1), jnp.float32),
        ],
        compiler_params=pltpu.CompilerParams(
            dimension_semantics=("arbitrary",),
            vmem_limit_bytes=120 * 1024 * 1024,
        ),
    )(adjacency_matrix, adjacency_matrix, features, W_ds, bds2, W0, R0, b02)

    return pl.pallas_call(
        _gc2_kernel,
        grid=(ni,),
        in_specs=[
            pl.BlockSpec((1, br, n), lambda i: (0, i, 0)),
            pl.BlockSpec((1, br, n), lambda i: (1, i, 0)),
            pl.BlockSpec((n, f1), lambda i: (0, 0)),
            pl.BlockSpec((1, f1), lambda i: (0, 0)),
            pl.BlockSpec((1, f1), lambda i: (0, 0)),
            pl.BlockSpec((t_count, f1, f2), lambda i: (0, 0, 0)),
            pl.BlockSpec((f1, f2), lambda i: (0, 0)),
            pl.BlockSpec((1, f2), lambda i: (0, 0)),
        ],
        out_specs=pl.BlockSpec((br, f2), lambda i: (i, 0)),
        out_shape=jax.ShapeDtypeStruct((n, f2), jnp.float32),
        scratch_shapes=[
            pltpu.VMEM((t_count, n, f2), jnp.bfloat16),
            pltpu.VMEM((n, f2), jnp.float32),
        ],
        compiler_params=pltpu.CompilerParams(
            dimension_semantics=("arbitrary",),
            vmem_limit_bytes=120 * 1024 * 1024,
        ),
    )(adjacency_matrix, adjacency_matrix, h1raw, g2, be2, W1, R1, b12)
